# baseline (speedup 1.0000x reference)
import functools

import jax
import jax.numpy as jnp
from jax import lax
from jax.experimental import pallas as pl
from jax.experimental.pallas import tpu as pltpu
from jax.experimental.pallas import tpu_sc as plsc

N = 10000
E = 320000
D = 128
R = 5

NC = 2
NS = 16
NW = NC * NS
L = 16

EPT = E // NW
B = 128
NB_FULL = EPT // B
TAIL = EPT - NB_FULL * B
E_PAD = NW * EPT + B

RN = R * N
RNP = 50176
DEG_TOTAL = 2 * RNP
TRASH = RN
ZROW = N

_mesh = plsc.VectorSubcoreMesh(core_axis_name="c", subcore_axis_name="s")


DEG_CHUNK = DEG_TOTAL // NS
DEG_NCOPY = DEG_CHUNK // B

DEG_K = 6
DEG_GRP = DEG_K * B
DEG_NGRP = NB_FULL // DEG_K


@functools.partial(
    pl.kernel,
    out_type=jax.ShapeDtypeStruct((NC, DEG_TOTAL), jnp.float32),
    mesh=_mesh,
    scratch_types=(
        [pltpu.VMEM((DEG_GRP,), jnp.int32) for _ in range(3)]
        + [pltpu.VMEM((B,), jnp.int32) for _ in range(2 * DEG_K)]
        + [
            pltpu.VMEM((B,), jnp.float32),
            pltpu.VMEM((B,), jnp.float32),
            pltpu.VMEM_SHARED((DEG_TOTAL,), jnp.float32),
            pltpu.SemaphoreType.DMA,
        ]
    ),
)
def _deg(src_hbm, dst_hbm, rate_hbm, out_hbm, sv_v, dv_v, rv_v,
         k0, k1, k2, k3, k4, k5, k6, k7, k8, k9, k10, k11,
         ones, zbuf, acc, sem):
    keys = [k0, k1, k2, k3, k4, k5, k6, k7, k8, k9, k10, k11]
    c = lax.axis_index("c")
    s = lax.axis_index("s")
    wid = s * NC + c

    z16 = jnp.zeros((L,), jnp.float32)
    o16 = jnp.ones((L,), jnp.float32)
    for j in range(B // L):
        sl = pl.ds(j * L, L)
        ones[sl] = o16
        zbuf[sl] = z16

    def zero_body(t, carry):
        pltpu.sync_copy(zbuf, acc.at[pl.ds(s * DEG_CHUNK + t * B, B)])
        return carry

    lax.fori_loop(0, DEG_NCOPY, zero_body, 0)
    plsc.subcore_barrier()

    base = wid * EPT

    def group_body(i, carry):
        off = base + i * DEG_GRP
        pltpu.sync_copy(src_hbm.at[pl.ds(off, DEG_GRP)], sv_v)
        pltpu.sync_copy(dst_hbm.at[pl.ds(off, DEG_GRP)], dv_v)
        pltpu.sync_copy(rate_hbm.at[pl.ds(off, DEG_GRP)], rv_v)
        for k in range(DEG_K):
            for j in range(B // L):
                gsl = pl.ds(k * B + j * L, L)
                sl = pl.ds(j * L, L)
                sv = sv_v[gsl]
                dv = dv_v[gsl]
                rv = rv_v[gsl]
                rn = rv * N
                keys[2 * k][sl] = rn + sv
                keys[2 * k + 1][sl] = rn + dv + RNP
        hs = [pltpu.async_copy(ones, acc.at[keys[q]], sem, add=True)
              for q in range(2 * DEG_K)]
        for h in hs:
            h.wait()
        return carry

    lax.fori_loop(0, DEG_NGRP, group_body, 0)

    off = base + DEG_NGRP * DEG_GRP
    pltpu.sync_copy(src_hbm.at[pl.ds(off, B)], sv_v.at[pl.ds(0, B)])
    pltpu.sync_copy(dst_hbm.at[pl.ds(off, B)], dv_v.at[pl.ds(0, B)])
    pltpu.sync_copy(rate_hbm.at[pl.ds(off, B)], rv_v.at[pl.ds(0, B)])
    for j in range(B // L):
        sl = pl.ds(j * L, L)
        if j * L >= TAIL:
            t16 = jnp.full((L,), TRASH, jnp.int32)
            keys[0][sl] = t16
            keys[1][sl] = t16
        else:
            sv = sv_v[sl]
            dv = dv_v[sl]
            rv = rv_v[sl]
            rn = rv * N
            keys[0][sl] = rn + sv
            keys[1][sl] = rn + dv + RNP
    pltpu.sync_copy(ones, acc.at[keys[0]], add=True)
    pltpu.sync_copy(ones, acc.at[keys[1]], add=True)

    plsc.subcore_barrier()

    def flush_body(t, carry):
        off2 = s * DEG_CHUNK + t * B
        pltpu.sync_copy(acc.at[pl.ds(off2, B)], zbuf)
        pltpu.sync_copy(zbuf, out_hbm.at[c, pl.ds(off2, B)])
        return carry

    lax.fori_loop(0, DEG_NCOPY, flush_body, 0)


HR = 1536
TRASH_ROW = HR
ACC_ROWS = HR + 8
NBANDS = 7
ROWS_PT = HR // NS
ZCH = 32

AB = 64
AK = 12
AGRP = AK * AB
ANG = EPT // AGRP


@functools.partial(
    pl.kernel,
    out_type=jax.ShapeDtypeStruct((NC, HR, D), jnp.float32),
    mesh=_mesh,
    scratch_types=(
        [pltpu.VMEM((AGRP,), jnp.int32) for _ in range(3)]
        + [pltpu.VMEM((L,), jnp.int32) for _ in range(2)]
        + [pltpu.VMEM((AB,), jnp.int32) for _ in range(AK)]
        + [pltpu.VMEM((AB,), jnp.int32) for _ in range(AK)]
        + [
            pltpu.VMEM((AK * AB, D), jnp.float32),
            pltpu.VMEM_SHARED((ACC_ROWS, D), jnp.float32),
            pltpu.SemaphoreType.DMA,
            pltpu.SemaphoreType.DMA,
        ]
    ),
)
def _agg(hsrc_hbm, src_hbm, dst_hbm, rate_hbm, rvec_hbm, bvec_hbm, out_hbm,
         sv_v, dv_v, rv_v, rv16, bv16,
         g0, g1, g2, g3, g4, g5, g6, g7, g8, g9, g10, g11,
         x0, x1, x2, x3, x4, x5, x6, x7, x8, x9, x10, x11,
         rows, acc, gsem, ssem):
    gidx = [g0, g1, g2, g3, g4, g5, g6, g7, g8, g9, g10, g11]
    six = [x0, x1, x2, x3, x4, x5, x6, x7, x8, x9, x10, x11]
    c = lax.axis_index("c")
    s = lax.axis_index("s")
    wid = s * NC + c

    pltpu.sync_copy(rvec_hbm, rv16)
    pltpu.sync_copy(bvec_hbm, bv16)
    rr = rv16[...]
    bb = bv16[...]

    zi16 = jnp.full((L,), ZROW, jnp.int32)
    for j in range(AB // L):
        gidx[0][pl.ds(j * L, L)] = zi16
    pltpu.sync_copy(hsrc_hbm.at[gidx[0]], rows.at[pl.ds(0, AB)])

    rbase = s * ROWS_PT
    for t in range(ROWS_PT // ZCH):
        pltpu.sync_copy(rows.at[pl.ds(0, ZCH)],
                        acc.at[pl.ds(rbase + t * ZCH, ZCH)])
    plsc.subcore_barrier()

    base = wid * EPT

    def group_body(i, carry):
        off = base + i * AGRP
        pltpu.sync_copy(src_hbm.at[pl.ds(off, AGRP)], sv_v)
        pltpu.sync_copy(dst_hbm.at[pl.ds(off, AGRP)], dv_v)
        pltpu.sync_copy(rate_hbm.at[pl.ds(off, AGRP)], rv_v)
        for k in range(AK):
            for j in range(AB // L):
                gsl = pl.ds(k * AB + j * L, L)
                sl = pl.ds(j * L, L)
                sv = sv_v[gsl]
                dv = dv_v[gsl]
                rv = rv_v[gsl]
                dl = dv - bb
                m = (rv == rr) & (dl >= 0) & (dl < HR)
                gidx[k][sl] = jnp.where(m, sv, ZROW)
                six[k][sl] = jnp.where(m, dl, TRASH_ROW)
        hs = [pltpu.async_copy(hsrc_hbm.at[gidx[k]],
                               rows.at[pl.ds(k * AB, AB)], gsem)
              for k in range(AK)]
        for h in hs:
            h.wait()
        hs = [pltpu.async_copy(rows.at[pl.ds(k * AB, AB)],
                               acc.at[six[k]], ssem, add=True)
              for k in range(AK)]
        for h in hs:
            h.wait()
        return carry

    lax.fori_loop(0, ANG, group_body, 0)

    off = base + ANG * AGRP
    pltpu.sync_copy(src_hbm.at[pl.ds(off, AB)], sv_v.at[pl.ds(0, AB)])
    pltpu.sync_copy(dst_hbm.at[pl.ds(off, AB)], dv_v.at[pl.ds(0, AB)])
    pltpu.sync_copy(rate_hbm.at[pl.ds(off, AB)], rv_v.at[pl.ds(0, AB)])
    for j in range(AB // L):
        sl = pl.ds(j * L, L)
        if j * L >= TAIL:
            gidx[0][sl] = jnp.full((L,), ZROW, jnp.int32)
            six[0][sl] = jnp.full((L,), TRASH_ROW, jnp.int32)
        else:
            sv = sv_v[sl]
            dv = dv_v[sl]
            rv = rv_v[sl]
            dl = dv - bb
            m = (rv == rr) & (dl >= 0) & (dl < HR)
            gidx[0][sl] = jnp.where(m, sv, ZROW)
            six[0][sl] = jnp.where(m, dl, TRASH_ROW)
    pltpu.sync_copy(hsrc_hbm.at[gidx[0]], rows.at[pl.ds(0, AB)])
    pltpu.sync_copy(rows.at[pl.ds(0, AB)], acc.at[six[0]], add=True)

    plsc.subcore_barrier()

    for t in range(ROWS_PT // ZCH):
        off2 = rbase + t * ZCH
        pltpu.sync_copy(acc.at[pl.ds(off2, ZCH)], rows.at[pl.ds(0, ZCH)])
        pltpu.sync_copy(rows.at[pl.ds(0, ZCH)],
                        out_hbm.at[c, pl.ds(off2, ZCH)])


def _norms_body(d_ref, o_ref):
    o_ref[...] = lax.rsqrt(jnp.maximum(d_ref[0] + d_ref[1], 1.0))


_norms_call = pl.pallas_call(
    _norms_body,
    out_shape=jax.ShapeDtypeStruct((DEG_TOTAL // D, D), jnp.float32),
)

BR = 1000


def _scale_body(x_ref, n_ref, o_ref):
    o_ref[...] = x_ref[...] * n_ref[...]


_scale_call = pl.pallas_call(
    _scale_body,
    grid=(N // BR,),
    in_specs=[
        pl.BlockSpec((BR, D), lambda i: (i, 0)),
        pl.BlockSpec((BR, 1), lambda i: (i, 0)),
    ],
    out_specs=pl.BlockSpec((BR, D), lambda i: (i, 0)),
    out_shape=jax.ShapeDtypeStruct((N, D), jnp.float32),
)


def _mm_body(p_ref, nd_ref, w_ref, b_ref, nn_ref, rst_ref, hn_ref):
    a = (p_ref[0] + p_ref[1]) * nd_ref[...]
    v = jnp.dot(a, w_ref[...], preferred_element_type=jnp.float32) + b_ref[...]
    rst_ref[...] = v
    hn_ref[...] = v * nn_ref[...]


_mm_call = pl.pallas_call(
    _mm_body,
    grid=(N // BR,),
    in_specs=[
        pl.BlockSpec((2, BR, D), lambda i: (0, i, 0)),
        pl.BlockSpec((BR, 1), lambda i: (i, 0)),
        pl.BlockSpec((D, D), lambda i: (0, 0)),
        pl.BlockSpec((1, D), lambda i: (0, 0)),
        pl.BlockSpec((BR, 1), lambda i: (i, 0)),
    ],
    out_specs=[
        pl.BlockSpec((BR, D), lambda i: (i, 0)),
        pl.BlockSpec((BR, D), lambda i: (i, 0)),
    ],
    out_shape=[
        jax.ShapeDtypeStruct((N, D), jnp.float32),
        jax.ShapeDtypeStruct((N, D), jnp.float32),
    ],
)


def kernel(x, edge_index, edge_rate, W, b):
    src = edge_index[0]
    dst = edge_index[1]
    pad = E_PAD - E
    srcp = jnp.pad(src, (0, pad))
    dstp = jnp.pad(dst, (0, pad))
    ratep = jnp.pad(edge_rate, (0, pad), constant_values=R)

    deg = _deg(srcp, dstp, ratep)
    norm = _norms_call(deg.reshape(NC, DEG_TOTAL // D, D)).reshape(-1)
    nsrc = norm[0:RN].reshape(R, N)
    ndst = norm[RNP:RNP + RN].reshape(R, N)

    ones_col = jnp.ones((N, 1), jnp.float32)
    rvecs = [jnp.full((L,), r, jnp.int32) for r in range(R)]
    bvecs = [jnp.full((L,), h * HR, jnp.int32) for h in range(NBANDS)]
    hsrc = _scale_call(x, nsrc[0].reshape(N, 1))
    outs = []
    for r in range(R):
        hsrc_pad = jnp.pad(hsrc, ((0, 1), (0, 0)))
        ps = [_agg(hsrc_pad, srcp, dstp, ratep, rvecs[r], bvecs[h])
              for h in range(NBANDS)]
        part = jnp.concatenate(ps, axis=1)
        nnext = nsrc[r + 1].reshape(N, 1) if r + 1 < R else ones_col
        rst, hsrc = _mm_call(part, ndst[r].reshape(N, 1), W[r],
                             b[r].reshape(1, D), nnext)
        outs.append(rst)
    return jnp.concatenate(outs, axis=1)


# speedup vs baseline: 1.7864x; 1.7864x over previous
import functools

import jax
import jax.numpy as jnp
from jax import lax
from jax.experimental import pallas as pl
from jax.experimental.pallas import tpu as pltpu
from jax.experimental.pallas import tpu_sc as plsc

N = 10000
E = 320000
D = 128
R = 5

NC = 2
NS = 16
NW = NC * NS
L = 16

EPT = E // NW
B = 128
NB_FULL = EPT // B
TAIL = EPT - NB_FULL * B
E_PAD = NW * EPT + B

RN = R * N
RNP = 50176
DEG_TOTAL = 2 * RNP
TRASH = RN
ZROW = N

_mesh = plsc.VectorSubcoreMesh(core_axis_name="c", subcore_axis_name="s")


DEG_CHUNK = DEG_TOTAL // NS
DEG_NCOPY = DEG_CHUNK // B

DEG_K = 6
DEG_GRP = DEG_K * B
DEG_NGRP = NB_FULL // DEG_K


@functools.partial(
    pl.kernel,
    out_type=jax.ShapeDtypeStruct((NC, DEG_TOTAL), jnp.float32),
    mesh=_mesh,
    scratch_types=(
        [pltpu.VMEM((DEG_GRP,), jnp.int32) for _ in range(3)]
        + [pltpu.VMEM((B,), jnp.int32) for _ in range(2 * DEG_K)]
        + [
            pltpu.VMEM((B,), jnp.float32),
            pltpu.VMEM((B,), jnp.float32),
            pltpu.VMEM_SHARED((DEG_TOTAL,), jnp.float32),
            pltpu.SemaphoreType.DMA,
        ]
    ),
)
def _deg(src_hbm, dst_hbm, rate_hbm, out_hbm, sv_v, dv_v, rv_v,
         k0, k1, k2, k3, k4, k5, k6, k7, k8, k9, k10, k11,
         ones, zbuf, acc, sem):
    keys = [k0, k1, k2, k3, k4, k5, k6, k7, k8, k9, k10, k11]
    c = lax.axis_index("c")
    s = lax.axis_index("s")
    wid = s * NC + c

    z16 = jnp.zeros((L,), jnp.float32)
    o16 = jnp.ones((L,), jnp.float32)
    for j in range(B // L):
        sl = pl.ds(j * L, L)
        ones[sl] = o16
        zbuf[sl] = z16

    def zero_body(t, carry):
        pltpu.sync_copy(zbuf, acc.at[pl.ds(s * DEG_CHUNK + t * B, B)])
        return carry

    lax.fori_loop(0, DEG_NCOPY, zero_body, 0)
    plsc.subcore_barrier()

    base = wid * EPT

    def group_body(i, carry):
        off = base + i * DEG_GRP
        pltpu.sync_copy(src_hbm.at[pl.ds(off, DEG_GRP)], sv_v)
        pltpu.sync_copy(dst_hbm.at[pl.ds(off, DEG_GRP)], dv_v)
        pltpu.sync_copy(rate_hbm.at[pl.ds(off, DEG_GRP)], rv_v)
        for k in range(DEG_K):
            for j in range(B // L):
                gsl = pl.ds(k * B + j * L, L)
                sl = pl.ds(j * L, L)
                sv = sv_v[gsl]
                dv = dv_v[gsl]
                rv = rv_v[gsl]
                rn = rv * N
                keys[2 * k][sl] = rn + sv
                keys[2 * k + 1][sl] = rn + dv + RNP
        hs = [pltpu.async_copy(ones, acc.at[keys[q]], sem, add=True)
              for q in range(2 * DEG_K)]
        for h in hs:
            h.wait()
        return carry

    lax.fori_loop(0, DEG_NGRP, group_body, 0)

    off = base + DEG_NGRP * DEG_GRP
    pltpu.sync_copy(src_hbm.at[pl.ds(off, B)], sv_v.at[pl.ds(0, B)])
    pltpu.sync_copy(dst_hbm.at[pl.ds(off, B)], dv_v.at[pl.ds(0, B)])
    pltpu.sync_copy(rate_hbm.at[pl.ds(off, B)], rv_v.at[pl.ds(0, B)])
    for j in range(B // L):
        sl = pl.ds(j * L, L)
        if j * L >= TAIL:
            t16 = jnp.full((L,), TRASH, jnp.int32)
            keys[0][sl] = t16
            keys[1][sl] = t16
        else:
            sv = sv_v[sl]
            dv = dv_v[sl]
            rv = rv_v[sl]
            rn = rv * N
            keys[0][sl] = rn + sv
            keys[1][sl] = rn + dv + RNP
    pltpu.sync_copy(ones, acc.at[keys[0]], add=True)
    pltpu.sync_copy(ones, acc.at[keys[1]], add=True)

    plsc.subcore_barrier()

    def flush_body(t, carry):
        off2 = s * DEG_CHUNK + t * B
        pltpu.sync_copy(acc.at[pl.ds(off2, B)], zbuf)
        pltpu.sync_copy(zbuf, out_hbm.at[c, pl.ds(off2, B)])
        return carry

    lax.fori_loop(0, DEG_NCOPY, flush_body, 0)


HR = 2500
TRASH_ROW = HR
ACC_ROWS = 2560
NBANDS = 4
ROWS_PT = ACC_ROWS // NS
ZCH = 32

AB = 64
AK = 12
AGRP = AK * AB
ANG = EPT // AGRP


@functools.partial(
    pl.kernel,
    out_type=jax.ShapeDtypeStruct((NC, ACC_ROWS, D), jnp.float32),
    mesh=_mesh,
    scratch_types=(
        [pltpu.VMEM((AGRP,), jnp.int32) for _ in range(3)]
        + [pltpu.VMEM((L,), jnp.int32) for _ in range(2)]
        + [pltpu.VMEM((AB,), jnp.int32) for _ in range(AK)]
        + [pltpu.VMEM((AB,), jnp.int32) for _ in range(AK)]
        + [
            pltpu.VMEM((AK * AB, D), jnp.float32),
            pltpu.VMEM_SHARED((ACC_ROWS, D), jnp.float32),
            pltpu.SemaphoreType.DMA,
            pltpu.SemaphoreType.DMA,
        ]
    ),
)
def _agg(hsrc_hbm, src_hbm, dst_hbm, rate_hbm, rvec_hbm, bvec_hbm, out_hbm,
         sv_v, dv_v, rv_v, rv16, bv16,
         g0, g1, g2, g3, g4, g5, g6, g7, g8, g9, g10, g11,
         x0, x1, x2, x3, x4, x5, x6, x7, x8, x9, x10, x11,
         rows, acc, gsem, ssem):
    gidx = [g0, g1, g2, g3, g4, g5, g6, g7, g8, g9, g10, g11]
    six = [x0, x1, x2, x3, x4, x5, x6, x7, x8, x9, x10, x11]
    c = lax.axis_index("c")
    s = lax.axis_index("s")
    wid = s * NC + c

    pltpu.sync_copy(rvec_hbm, rv16)
    pltpu.sync_copy(bvec_hbm, bv16)
    rr = rv16[...]
    bb = bv16[...]

    zi16 = jnp.full((L,), ZROW, jnp.int32)
    for j in range(AB // L):
        gidx[0][pl.ds(j * L, L)] = zi16
    pltpu.sync_copy(hsrc_hbm.at[gidx[0]], rows.at[pl.ds(0, AB)])

    rbase = s * ROWS_PT

    def zero_body(t, carry):
        pltpu.sync_copy(rows.at[pl.ds(0, ZCH)],
                        acc.at[pl.ds(rbase + t * ZCH, ZCH)])
        return carry

    lax.fori_loop(0, ROWS_PT // ZCH, zero_body, 0)
    plsc.subcore_barrier()

    base = wid * EPT

    def group_body(i, carry):
        off = base + i * AGRP
        pltpu.sync_copy(src_hbm.at[pl.ds(off, AGRP)], sv_v)
        pltpu.sync_copy(dst_hbm.at[pl.ds(off, AGRP)], dv_v)
        pltpu.sync_copy(rate_hbm.at[pl.ds(off, AGRP)], rv_v)
        for k in range(AK):
            for j in range(AB // L):
                gsl = pl.ds(k * AB + j * L, L)
                sl = pl.ds(j * L, L)
                sv = sv_v[gsl]
                dv = dv_v[gsl]
                rv = rv_v[gsl]
                dl = dv - bb
                m = (rv == rr) & (dl >= 0) & (dl < HR)
                gidx[k][sl] = jnp.where(m, sv, ZROW)
                six[k][sl] = jnp.where(m, dl, TRASH_ROW)
        hs = [pltpu.async_copy(hsrc_hbm.at[gidx[k]],
                               rows.at[pl.ds(k * AB, AB)], gsem)
              for k in range(AK)]
        ss = []
        for k in range(AK):
            hs[k].wait()
            ss.append(pltpu.async_copy(rows.at[pl.ds(k * AB, AB)],
                                       acc.at[six[k]], ssem, add=True))
        for h in ss:
            h.wait()
        return carry

    lax.fori_loop(0, ANG, group_body, 0)

    off = base + ANG * AGRP
    pltpu.sync_copy(src_hbm.at[pl.ds(off, AB)], sv_v.at[pl.ds(0, AB)])
    pltpu.sync_copy(dst_hbm.at[pl.ds(off, AB)], dv_v.at[pl.ds(0, AB)])
    pltpu.sync_copy(rate_hbm.at[pl.ds(off, AB)], rv_v.at[pl.ds(0, AB)])
    for j in range(AB // L):
        sl = pl.ds(j * L, L)
        if j * L >= TAIL:
            gidx[0][sl] = jnp.full((L,), ZROW, jnp.int32)
            six[0][sl] = jnp.full((L,), TRASH_ROW, jnp.int32)
        else:
            sv = sv_v[sl]
            dv = dv_v[sl]
            rv = rv_v[sl]
            dl = dv - bb
            m = (rv == rr) & (dl >= 0) & (dl < HR)
            gidx[0][sl] = jnp.where(m, sv, ZROW)
            six[0][sl] = jnp.where(m, dl, TRASH_ROW)
    pltpu.sync_copy(hsrc_hbm.at[gidx[0]], rows.at[pl.ds(0, AB)])
    pltpu.sync_copy(rows.at[pl.ds(0, AB)], acc.at[six[0]], add=True)

    plsc.subcore_barrier()

    def flush_body(t, carry):
        off2 = rbase + t * ZCH
        pltpu.sync_copy(acc.at[pl.ds(off2, ZCH)], rows.at[pl.ds(0, ZCH)])
        pltpu.sync_copy(rows.at[pl.ds(0, ZCH)],
                        out_hbm.at[c, pl.ds(off2, ZCH)])
        return carry

    lax.fori_loop(0, ROWS_PT // ZCH, flush_body, 0)


def _norms_body(d_ref, o_ref):
    o_ref[...] = lax.rsqrt(jnp.maximum(d_ref[0] + d_ref[1], 1.0))


_norms_call = pl.pallas_call(
    _norms_body,
    out_shape=jax.ShapeDtypeStruct((DEG_TOTAL // D, D), jnp.float32),
)

BR = 1000


def _scale_body(x_ref, n_ref, o_ref):
    o_ref[...] = x_ref[...] * n_ref[...]


_scale_call = pl.pallas_call(
    _scale_body,
    grid=(N // BR,),
    in_specs=[
        pl.BlockSpec((BR, D), lambda i: (i, 0)),
        pl.BlockSpec((BR, 1), lambda i: (i, 0)),
    ],
    out_specs=pl.BlockSpec((BR, D), lambda i: (i, 0)),
    out_shape=jax.ShapeDtypeStruct((N, D), jnp.float32),
)


def _mm_body(p_ref, nd_ref, w_ref, b_ref, nn_ref, rst_ref, hn_ref):
    a = (p_ref[0] + p_ref[1]) * nd_ref[...]
    v = jnp.dot(a, w_ref[...], preferred_element_type=jnp.float32) + b_ref[...]
    rst_ref[...] = v
    hn_ref[...] = v * nn_ref[...]


_mm_call = pl.pallas_call(
    _mm_body,
    grid=(N // BR,),
    in_specs=[
        pl.BlockSpec((2, BR, D), lambda i: (0, i, 0)),
        pl.BlockSpec((BR, 1), lambda i: (i, 0)),
        pl.BlockSpec((D, D), lambda i: (0, 0)),
        pl.BlockSpec((1, D), lambda i: (0, 0)),
        pl.BlockSpec((BR, 1), lambda i: (i, 0)),
    ],
    out_specs=[
        pl.BlockSpec((BR, D), lambda i: (i, 0)),
        pl.BlockSpec((BR, D), lambda i: (i, 0)),
    ],
    out_shape=[
        jax.ShapeDtypeStruct((N, D), jnp.float32),
        jax.ShapeDtypeStruct((N, D), jnp.float32),
    ],
)


def kernel(x, edge_index, edge_rate, W, b):
    src = edge_index[0]
    dst = edge_index[1]
    pad = E_PAD - E
    srcp = jnp.pad(src, (0, pad))
    dstp = jnp.pad(dst, (0, pad))
    ratep = jnp.pad(edge_rate, (0, pad), constant_values=R)

    deg = _deg(srcp, dstp, ratep)
    norm = _norms_call(deg.reshape(NC, DEG_TOTAL // D, D)).reshape(-1)
    nsrc = norm[0:RN].reshape(R, N)
    ndst = norm[RNP:RNP + RN].reshape(R, N)

    ones_col = jnp.ones((N, 1), jnp.float32)
    rvecs = [jnp.full((L,), r, jnp.int32) for r in range(R)]
    bvecs = [jnp.full((L,), h * HR, jnp.int32) for h in range(NBANDS)]
    hsrc = _scale_call(x, nsrc[0].reshape(N, 1))
    outs = []
    for r in range(R):
        hsrc_pad = jnp.pad(hsrc, ((0, 1), (0, 0)))
        parts = []
        tok = jnp.int32(0)
        for h in range(NBANDS):
            p = _agg(hsrc_pad, srcp, dstp, ratep, rvecs[r], bvecs[h] + tok)
            tok = (p[0, 0, 0] * 0.0).astype(jnp.int32)
            parts.append(p[:, :HR, :])
        pfull = jnp.concatenate(parts, axis=1)
        nnext = nsrc[r + 1].reshape(N, 1) if r + 1 < R else ones_col
        rst, hsrc = _mm_call(pfull, ndst[r].reshape(N, 1), W[r],
                             b[r].reshape(1, D), nnext)
        outs.append(rst)
    return jnp.concatenate(outs, axis=1)


# speedup vs baseline: 1.7867x; 1.0002x over previous
import functools

import jax
import jax.numpy as jnp
from jax import lax
from jax.experimental import pallas as pl
from jax.experimental.pallas import tpu as pltpu
from jax.experimental.pallas import tpu_sc as plsc

N = 10000
E = 320000
D = 128
R = 5

NC = 2
NS = 16
NW = NC * NS
L = 16

EPT = E // NW
B = 128
NB_FULL = EPT // B
TAIL = EPT - NB_FULL * B
E_PAD = NW * EPT + B

RN = R * N
RNP = 50176
DEG_TOTAL = 2 * RNP
TRASH = RN
ZROW = N

_mesh = plsc.VectorSubcoreMesh(core_axis_name="c", subcore_axis_name="s")


DEG_CHUNK = DEG_TOTAL // NS
DEG_NCOPY = DEG_CHUNK // B

DEG_K = 6
DEG_GRP = DEG_K * B
DEG_NGRP = NB_FULL // DEG_K


@functools.partial(
    pl.kernel,
    out_type=jax.ShapeDtypeStruct((NC, DEG_TOTAL), jnp.float32),
    mesh=_mesh,
    scratch_types=(
        [pltpu.VMEM((DEG_GRP,), jnp.int32) for _ in range(3)]
        + [pltpu.VMEM((B,), jnp.int32) for _ in range(2 * DEG_K)]
        + [
            pltpu.VMEM((B,), jnp.float32),
            pltpu.VMEM((B,), jnp.float32),
            pltpu.VMEM_SHARED((DEG_TOTAL,), jnp.float32),
            pltpu.SemaphoreType.DMA,
        ]
    ),
)
def _deg(src_hbm, dst_hbm, rate_hbm, out_hbm, sv_v, dv_v, rv_v,
         k0, k1, k2, k3, k4, k5, k6, k7, k8, k9, k10, k11,
         ones, zbuf, acc, sem):
    keys = [k0, k1, k2, k3, k4, k5, k6, k7, k8, k9, k10, k11]
    c = lax.axis_index("c")
    s = lax.axis_index("s")
    wid = s * NC + c

    z16 = jnp.zeros((L,), jnp.float32)
    o16 = jnp.ones((L,), jnp.float32)
    for j in range(B // L):
        sl = pl.ds(j * L, L)
        ones[sl] = o16
        zbuf[sl] = z16

    def zero_body(t, carry):
        pltpu.sync_copy(zbuf, acc.at[pl.ds(s * DEG_CHUNK + t * B, B)])
        return carry

    lax.fori_loop(0, DEG_NCOPY, zero_body, 0)
    plsc.subcore_barrier()

    base = wid * EPT

    def group_body(i, carry):
        off = base + i * DEG_GRP
        pltpu.sync_copy(src_hbm.at[pl.ds(off, DEG_GRP)], sv_v)
        pltpu.sync_copy(dst_hbm.at[pl.ds(off, DEG_GRP)], dv_v)
        pltpu.sync_copy(rate_hbm.at[pl.ds(off, DEG_GRP)], rv_v)
        for k in range(DEG_K):
            for j in range(B // L):
                gsl = pl.ds(k * B + j * L, L)
                sl = pl.ds(j * L, L)
                sv = sv_v[gsl]
                dv = dv_v[gsl]
                rv = rv_v[gsl]
                rn = rv * N
                keys[2 * k][sl] = rn + sv
                keys[2 * k + 1][sl] = rn + dv + RNP
        hs = [pltpu.async_copy(ones, acc.at[keys[q]], sem, add=True)
              for q in range(2 * DEG_K)]
        for h in hs:
            h.wait()
        return carry

    lax.fori_loop(0, DEG_NGRP, group_body, 0)

    off = base + DEG_NGRP * DEG_GRP
    pltpu.sync_copy(src_hbm.at[pl.ds(off, B)], sv_v.at[pl.ds(0, B)])
    pltpu.sync_copy(dst_hbm.at[pl.ds(off, B)], dv_v.at[pl.ds(0, B)])
    pltpu.sync_copy(rate_hbm.at[pl.ds(off, B)], rv_v.at[pl.ds(0, B)])
    for j in range(B // L):
        sl = pl.ds(j * L, L)
        if j * L >= TAIL:
            t16 = jnp.full((L,), TRASH, jnp.int32)
            keys[0][sl] = t16
            keys[1][sl] = t16
        else:
            sv = sv_v[sl]
            dv = dv_v[sl]
            rv = rv_v[sl]
            rn = rv * N
            keys[0][sl] = rn + sv
            keys[1][sl] = rn + dv + RNP
    pltpu.sync_copy(ones, acc.at[keys[0]], add=True)
    pltpu.sync_copy(ones, acc.at[keys[1]], add=True)

    plsc.subcore_barrier()

    def flush_body(t, carry):
        off2 = s * DEG_CHUNK + t * B
        pltpu.sync_copy(acc.at[pl.ds(off2, B)], zbuf)
        pltpu.sync_copy(zbuf, out_hbm.at[c, pl.ds(off2, B)])
        return carry

    lax.fori_loop(0, DEG_NCOPY, flush_body, 0)


HR = 2500
TRASH_ROW = HR
ACC_ROWS = 2560
NBANDS = 4
ROWS_PT = ACC_ROWS // NS
ZCH = 32

AB = 64
AK = 12
AGRP = AK * AB
ANG = EPT // AGRP


@functools.partial(
    pl.kernel,
    out_type=jax.ShapeDtypeStruct((NC, ACC_ROWS, D), jnp.float32),
    mesh=_mesh,
    scratch_types=(
        [pltpu.VMEM((AGRP,), jnp.int32) for _ in range(3)]
        + [pltpu.VMEM((L,), jnp.int32) for _ in range(2)]
        + [pltpu.VMEM((AB,), jnp.int32) for _ in range(AK)]
        + [pltpu.VMEM((AB,), jnp.int32) for _ in range(AK)]
        + [
            pltpu.VMEM((AK * AB, D), jnp.float32),
            pltpu.VMEM_SHARED((ACC_ROWS, D), jnp.float32),
            pltpu.SemaphoreType.DMA,
            pltpu.SemaphoreType.DMA,
        ]
    ),
)
def _agg(hsrc_hbm, src_hbm, dst_hbm, rate_hbm, rvec_hbm, bvec_hbm, out_hbm,
         sv_v, dv_v, rv_v, rv16, bv16,
         g0, g1, g2, g3, g4, g5, g6, g7, g8, g9, g10, g11,
         x0, x1, x2, x3, x4, x5, x6, x7, x8, x9, x10, x11,
         rows, acc, gsem, ssem):
    gidx = [g0, g1, g2, g3, g4, g5, g6, g7, g8, g9, g10, g11]
    six = [x0, x1, x2, x3, x4, x5, x6, x7, x8, x9, x10, x11]
    c = lax.axis_index("c")
    s = lax.axis_index("s")
    wid = s * NC + c

    pltpu.sync_copy(rvec_hbm, rv16)
    pltpu.sync_copy(bvec_hbm, bv16)
    rr = rv16[...]
    bb = bv16[...]

    zi16 = jnp.full((L,), ZROW, jnp.int32)
    for j in range(AB // L):
        gidx[0][pl.ds(j * L, L)] = zi16
    pltpu.sync_copy(hsrc_hbm.at[gidx[0]], rows.at[pl.ds(0, AB)])

    rbase = s * ROWS_PT

    def zero_body(t, carry):
        pltpu.sync_copy(rows.at[pl.ds(0, ZCH)],
                        acc.at[pl.ds(rbase + t * ZCH, ZCH)])
        return carry

    lax.fori_loop(0, ROWS_PT // ZCH, zero_body, 0)
    plsc.subcore_barrier()

    base = wid * EPT

    def group_body(i, carry):
        off = base + i * AGRP
        pltpu.sync_copy(src_hbm.at[pl.ds(off, AGRP)], sv_v)
        pltpu.sync_copy(dst_hbm.at[pl.ds(off, AGRP)], dv_v)
        pltpu.sync_copy(rate_hbm.at[pl.ds(off, AGRP)], rv_v)
        for k in range(AK):
            for j in range(AB // L):
                gsl = pl.ds(k * AB + j * L, L)
                sl = pl.ds(j * L, L)
                sv = sv_v[gsl]
                dv = dv_v[gsl]
                rv = rv_v[gsl]
                dl = dv - bb
                m = (rv == rr) & (dl >= 0) & (dl < HR)
                gidx[k][sl] = jnp.where(m, sv, ZROW)
                six[k][sl] = jnp.where(m, dl, TRASH_ROW)
        hs = [pltpu.async_copy(hsrc_hbm.at[gidx[k]],
                               rows.at[pl.ds(k * AB, AB)], gsem)
              for k in range(AK)]
        ss = []
        for k in range(AK):
            hs[k].wait()
            ss.append(pltpu.async_copy(rows.at[pl.ds(k * AB, AB)],
                                       acc.at[six[k]], ssem, add=True))
        for h in ss:
            h.wait()
        return carry

    lax.fori_loop(0, ANG, group_body, 0)

    off = base + ANG * AGRP
    pltpu.sync_copy(src_hbm.at[pl.ds(off, AB)], sv_v.at[pl.ds(0, AB)])
    pltpu.sync_copy(dst_hbm.at[pl.ds(off, AB)], dv_v.at[pl.ds(0, AB)])
    pltpu.sync_copy(rate_hbm.at[pl.ds(off, AB)], rv_v.at[pl.ds(0, AB)])
    for j in range(AB // L):
        sl = pl.ds(j * L, L)
        if j * L >= TAIL:
            gidx[0][sl] = jnp.full((L,), ZROW, jnp.int32)
            six[0][sl] = jnp.full((L,), TRASH_ROW, jnp.int32)
        else:
            sv = sv_v[sl]
            dv = dv_v[sl]
            rv = rv_v[sl]
            dl = dv - bb
            m = (rv == rr) & (dl >= 0) & (dl < HR)
            gidx[0][sl] = jnp.where(m, sv, ZROW)
            six[0][sl] = jnp.where(m, dl, TRASH_ROW)
    pltpu.sync_copy(hsrc_hbm.at[gidx[0]], rows.at[pl.ds(0, AB)])
    pltpu.sync_copy(rows.at[pl.ds(0, AB)], acc.at[six[0]], add=True)

    plsc.subcore_barrier()

    def flush_body(t, carry):
        off2 = rbase + t * ZCH
        pltpu.sync_copy(acc.at[pl.ds(off2, ZCH)], rows.at[pl.ds(0, ZCH)])
        pltpu.sync_copy(rows.at[pl.ds(0, ZCH)],
                        out_hbm.at[c, pl.ds(off2, ZCH)])
        return carry

    lax.fori_loop(0, ROWS_PT // ZCH, flush_body, 0)


def _norms_body(d_ref, o_ref):
    o_ref[...] = lax.rsqrt(jnp.maximum(d_ref[0] + d_ref[1], 1.0))


_norms_call = pl.pallas_call(
    _norms_body,
    out_shape=jax.ShapeDtypeStruct((DEG_TOTAL // D, D), jnp.float32),
)

BR = 1000


def _scale_body(x_ref, n_ref, o_ref):
    o_ref[...] = x_ref[...] * n_ref[...]


_scale_call = pl.pallas_call(
    _scale_body,
    grid=(N // BR,),
    in_specs=[
        pl.BlockSpec((BR, D), lambda i: (i, 0)),
        pl.BlockSpec((BR, 1), lambda i: (i, 0)),
    ],
    out_specs=pl.BlockSpec((BR, D), lambda i: (i, 0)),
    out_shape=jax.ShapeDtypeStruct((N, D), jnp.float32),
)


def _mm_body(p_ref, nd_ref, w_ref, b_ref, nn_ref, rst_ref, hn_ref):
    a = (p_ref[0] + p_ref[1]) * nd_ref[...]
    v = jnp.dot(a, w_ref[...], preferred_element_type=jnp.float32) + b_ref[...]
    rst_ref[...] = v
    hn_ref[...] = v * nn_ref[...]


_mm_call = pl.pallas_call(
    _mm_body,
    grid=(N // BR,),
    in_specs=[
        pl.BlockSpec((2, BR, D), lambda i: (0, i, 0)),
        pl.BlockSpec((BR, 1), lambda i: (i, 0)),
        pl.BlockSpec((D, D), lambda i: (0, 0)),
        pl.BlockSpec((1, D), lambda i: (0, 0)),
        pl.BlockSpec((BR, 1), lambda i: (i, 0)),
    ],
    out_specs=[
        pl.BlockSpec((BR, D), lambda i: (i, 0)),
        pl.BlockSpec((BR, D), lambda i: (i, 0)),
    ],
    out_shape=[
        jax.ShapeDtypeStruct((N, D), jnp.float32),
        jax.ShapeDtypeStruct((N, D), jnp.float32),
    ],
)


def kernel(x, edge_index, edge_rate, W, b):
    src = edge_index[0]
    dst = edge_index[1]
    pad = E_PAD - E
    srcp = jnp.pad(src, (0, pad))
    dstp = jnp.pad(dst, (0, pad))
    ratep = jnp.pad(edge_rate, (0, pad), constant_values=R)

    deg = _deg(srcp, dstp, ratep)
    norm = _norms_call(deg.reshape(NC, DEG_TOTAL // D, D)).reshape(-1)
    nsrc = norm[0:RN].reshape(R, N)
    ndst = norm[RNP:RNP + RN].reshape(R, N)

    ones_col = jnp.ones((N, 1), jnp.float32)
    rvecs = [jnp.full((L,), r, jnp.int32) for r in range(R)]
    bvecs = [jnp.full((L,), h * HR, jnp.int32) for h in range(NBANDS)]
    hsrc = _scale_call(x, nsrc[0].reshape(N, 1))
    outs = []
    for r in range(R):
        hsrc_pad = jnp.pad(hsrc, ((0, 1), (0, 0)))
        parts = []
        for h in range(NBANDS):
            p = _agg(hsrc_pad, srcp, dstp, ratep, rvecs[r], bvecs[h])
            parts.append(p[:, :HR, :])
        pfull = jnp.concatenate(parts, axis=1)
        nnext = nsrc[r + 1].reshape(N, 1) if r + 1 < R else ones_col
        rst, hsrc = _mm_call(pfull, ndst[r].reshape(N, 1), W[r],
                             b[r].reshape(1, D), nnext)
        outs.append(rst)
    return jnp.concatenate(outs, axis=1)
